# baseline (speedup 1.0000x reference)
import functools

import jax
import jax.numpy as jnp
from jax import lax
from jax.experimental import pallas as pl
from jax.experimental.pallas import tpu as pltpu
from jax.experimental.pallas import tpu_sc as plsc

_mesh = plsc.VectorSubcoreMesh(core_axis_name="c", subcore_axis_name="s")


@functools.partial(
    pl.kernel,
    mesh=_mesh,
    out_type=jax.ShapeDtypeStruct((3, 128), jnp.float32),
)
def _gather_rows(x_hbm, out_hbm):
    wid = lax.axis_index("s") * 2 + lax.axis_index("c")

    @pl.when(wid == 0)
    def _():
        pltpu.sync_copy(x_hbm.at[pl.ds(2, 1)], out_hbm.at[pl.ds(0, 1)])
        pltpu.sync_copy(x_hbm.at[pl.ds(4, 2)], out_hbm.at[pl.ds(1, 2)])


def kernel(x):
    return _gather_rows(x)


# speedup vs baseline: 1.1862x; 1.1862x over previous
import functools

import jax
import jax.numpy as jnp
from jax import lax
from jax.experimental import pallas as pl
from jax.experimental.pallas import tpu as pltpu
from jax.experimental.pallas import tpu_sc as plsc

_mesh = plsc.ScalarSubcoreMesh(axis_name="c", num_cores=1)


@functools.partial(
    pl.kernel,
    mesh=_mesh,
    out_type=jax.ShapeDtypeStruct((3, 128), jnp.float32),
)
def _gather_rows(x_hbm, out_hbm):
    pltpu.sync_copy(x_hbm.at[pl.ds(2, 1)], out_hbm.at[pl.ds(0, 1)])
    pltpu.sync_copy(x_hbm.at[pl.ds(4, 2)], out_hbm.at[pl.ds(1, 2)])


def kernel(x):
    return _gather_rows(x)


# speedup vs baseline: 1.2412x; 1.0464x over previous
import functools

import jax
import jax.numpy as jnp
from jax import lax
from jax.experimental import pallas as pl
from jax.experimental.pallas import tpu as pltpu
from jax.experimental.pallas import tpu_sc as plsc

_mesh = plsc.ScalarSubcoreMesh(axis_name="c", num_cores=1)


@functools.partial(
    pl.kernel,
    mesh=_mesh,
    out_type=jax.ShapeDtypeStruct((3, 128), jnp.float32),
    scratch_types=[pltpu.SemaphoreType.DMA, pltpu.SemaphoreType.DMA],
)
def _gather_rows(x_hbm, out_hbm, sem1, sem2):
    c1 = pltpu.async_copy(x_hbm.at[pl.ds(2, 1)], out_hbm.at[pl.ds(0, 1)], sem1)
    c2 = pltpu.async_copy(x_hbm.at[pl.ds(4, 2)], out_hbm.at[pl.ds(1, 2)], sem2)
    c1.wait()
    c2.wait()


def kernel(x):
    return _gather_rows(x)


# speedup vs baseline: 1.2428x; 1.0013x over previous
import functools

import jax
import jax.numpy as jnp
from jax import lax
from jax.experimental import pallas as pl
from jax.experimental.pallas import tpu as pltpu
from jax.experimental.pallas import tpu_sc as plsc

_mesh = plsc.ScalarSubcoreMesh(axis_name="c", num_cores=1)


@functools.partial(
    pl.kernel,
    mesh=_mesh,
    out_type=jax.ShapeDtypeStruct((3, 128), jnp.float32),
    scratch_types=[pltpu.SemaphoreType.DMA],
)
def _gather_rows(x_hbm, out_hbm, sem):
    c2 = pltpu.async_copy(x_hbm.at[pl.ds(4, 2)], out_hbm.at[pl.ds(1, 2)], sem)
    c1 = pltpu.async_copy(x_hbm.at[pl.ds(2, 1)], out_hbm.at[pl.ds(0, 1)], sem)
    c2.wait()
    c1.wait()


def kernel(x):
    return _gather_rows(x)
